# Initial kernel scaffold; baseline (speedup 1.0000x reference)
import jax
import jax.numpy as jnp
from jax.experimental import pallas as pl


def kernel(x, edge_index, batch, W_emb, b_emb, c1_W1, c1_b1, c1_W2, c1_b2, c2_W1, c2_b1, c2_W2, c2_b2):
    raise NotImplementedError("write your pallas kernel here")


# speedup vs baseline: 1.8757x; 1.8757x over previous
import functools

import jax
import jax.numpy as jnp
from jax import lax
from jax.experimental import pallas as pl
from jax.experimental.pallas import tpu as pltpu
from jax.experimental.pallas import tpu_sc as plsc

N_NODES = 10000
N_EDGES = 320000
HID = 128
N_GRAPHS = 16

NC = 2
NS = 16
NW = NC * NS
RANGE = 313
N_PAD = NW * RANGE
SHIFT = 1 << 14
PADVAL = RANGE * SHIFT

CH_A = 2000
NCH_A = N_EDGES // CH_A
FLUSH = 63488
BUFCAP = 66048
LISTCAP = 322176
CH_B = 128

_SC_PARAMS = pltpu.CompilerParams(needs_layout_passes=False)


def _wid():
    return lax.axis_index("s") * NC + lax.axis_index("c")


def _partition_body(src_hbm, dst_hbm, lists_hbm, counts_hbm, src_v, dst_v, outbuf, cnt_stage):
    wid = _wid()
    lo = wid * RANGE
    hi = lo + RANGE
    lbase = wid * LISTCAP

    def group(j, cnt):
        s = src_v[pl.ds(j * 16, 16)]
        t = dst_v[pl.ds(j * 16, 16)]
        msk = (t >= lo) & (t < hi)
        packed = s + (t - lo) * SHIFT
        cs = plsc.cumsum(msk.astype(jnp.int32))
        plsc.store_scatter(outbuf, [cnt + cs - 1], packed, mask=msk)
        return cnt + cs[15]

    def do_flush(cnt, written):
        pltpu.sync_copy(outbuf.at[pl.ds(0, FLUSH)],
                        lists_hbm.at[pl.ds(pl.multiple_of(lbase + written, 2048), FLUSH)])
        for j in range(128):
            outbuf[pl.ds(j * 16, 16)] = outbuf[pl.ds(FLUSH + j * 16, 16)]
        return cnt - FLUSH, written + FLUSH

    def no_flush(cnt, written):
        return cnt, written

    def chunk(i, carry):
        cnt, written = carry
        off = i * CH_A
        pltpu.sync_copy(src_hbm.at[pl.ds(pl.multiple_of(off, 8), CH_A)], src_v)
        pltpu.sync_copy(dst_hbm.at[pl.ds(pl.multiple_of(off, 8), CH_A)], dst_v)
        cnt = lax.fori_loop(0, CH_A // 16, group, cnt)
        cnt, written = lax.cond(cnt >= FLUSH, do_flush, no_flush, cnt, written)
        return cnt, written

    cnt, written = lax.fori_loop(0, NCH_A, chunk, (jnp.int32(0), jnp.int32(0)))

    padv = jnp.full((16,), PADVAL, jnp.int32)
    for j in range(8):
        outbuf[pl.ds(cnt + j * 16, 16)] = padv
    cnt_p = ((cnt + CH_B - 1) // CH_B) * CH_B

    def blk(b, _):
        pltpu.sync_copy(outbuf.at[pl.ds(b * 2048, 2048)],
                        lists_hbm.at[pl.ds(pl.multiple_of(lbase + written + b * 2048, 2048), 2048)])
        return 0

    lax.fori_loop(0, (cnt_p + 2047) // 2048, blk, 0)
    total = written + cnt_p
    cnt_stage[...] = jnp.broadcast_to(total, (16,))
    pltpu.sync_copy(cnt_stage, counts_hbm.at[pl.ds(pl.multiple_of(wid * 16, 16), 16)])


def _segmax_body(m_hbm, lists_hbm, counts_hbm, agg_hbm,
            acc, packed_v, src_v, rows, cnt_stage, sem):
    wid = _wid()
    lo = wid * RANGE

    pltpu.sync_copy(counts_hbm.at[pl.ds(pl.multiple_of(wid * 16, 16), 16)], cnt_stage)
    total = cnt_stage[pl.ds(0, 16)][0]

    ninf = jnp.full((16,), -jnp.inf, jnp.float32)

    def init(i, _):
        acc[pl.ds(i * 16, 16)] = ninf
        return 0

    lax.fori_loop(0, (RANGE + 1) * HID // 16, init, 0)

    def chunk(ch, _):
        pltpu.sync_copy(lists_hbm.at[pl.ds(pl.multiple_of(wid * LISTCAP + ch * CH_B, CH_B), CH_B)],
                        packed_v.at[pl.ds(0, CH_B)])
        for j in range(CH_B // 16):
            pv = packed_v[pl.ds(j * 16, 16)]
            src_v[pl.ds(j * 16, 16)] = pv & (SHIFT - 1)
        pltpu.async_copy(m_hbm.at[src_v], rows, sem).wait()

        def edge(e, _):
            pk = packed_v[pl.ds(e, 16)][0]
            base = (pk >> 14) * HID
            for c in range(HID // 16):
                off = base + c * 16
                acc[pl.ds(off, 16)] = jnp.maximum(acc[pl.ds(off, 16)],
                                                  rows[e, pl.ds(c * 16, 16)])
            return 0

        lax.fori_loop(0, CH_B, edge, 0)
        return 0

    lax.fori_loop(0, total // CH_B, chunk, 0)

    def fix(i, _):
        v = acc[pl.ds(i * 16, 16)]
        acc[pl.ds(i * 16, 16)] = jnp.where(v == -jnp.inf, 0.0, v)
        return 0

    lax.fori_loop(0, RANGE * HID // 16, fix, 0)
    pltpu.sync_copy(acc.at[pl.ds(0, RANGE * HID)],
                    agg_hbm.at[pl.ds(pl.multiple_of(lo * HID, 8), RANGE * HID)])


def _silu(x):
    return x * (1.0 / (1.0 + jnp.exp(-x)))


def _mlp(h, w1, b1, w2, b2):
    t = jnp.maximum(jnp.dot(h, w1, preferred_element_type=jnp.float32) + b1, 0.0)
    return jnp.dot(t, w2, preferred_element_type=jnp.float32) + b2


def _embed_body(x_ref, wemb_ref, bemb_ref, w1_ref, b1_ref, w2_ref, b2_ref,
                h0_ref, m1_ref):
    h0 = _silu(x_ref[...] * wemb_ref[...] + bemb_ref[...])
    h0_ref[...] = h0
    m1_ref[...] = _mlp(h0, w1_ref[...], b1_ref[...], w2_ref[...], b2_ref[...])


def _combine_body(h_ref, agg_ref, w1_ref, b1_ref, w2_ref, b2_ref,
                  h1_ref, m2_ref):
    h1 = h_ref[...] + _silu(agg_ref[...])
    h1_ref[...] = h1
    m2_ref[...] = _mlp(h1, w1_ref[...], b1_ref[...], w2_ref[...], b2_ref[...])


def _final_body(h_ref, agg_ref, batch_ref, out_ref):
    h2 = h_ref[...] + _silu(agg_ref[...])
    gid = lax.broadcasted_iota(jnp.int32, (N_GRAPHS, N_NODES), 0)
    oh = (gid == batch_ref[...]).astype(jnp.float32)
    sums = jnp.dot(oh, h2, preferred_element_type=jnp.float32)
    counts = jnp.sum(oh, axis=1, keepdims=True)
    out_ref[...] = sums / jnp.maximum(counts, 1.0)


_F32 = jnp.float32

_embed = pl.pallas_call(
    _embed_body,
    out_shape=[jax.ShapeDtypeStruct((N_NODES, HID), _F32),
               jax.ShapeDtypeStruct((N_NODES, HID), _F32)],
)

_combine = pl.pallas_call(
    _combine_body,
    out_shape=[jax.ShapeDtypeStruct((N_NODES, HID), _F32),
               jax.ShapeDtypeStruct((N_NODES, HID), _F32)],
)

_final = pl.pallas_call(
    _final_body,
    out_shape=jax.ShapeDtypeStruct((N_GRAPHS, HID), _F32),
)


@functools.lru_cache(maxsize=1)
def _sc_kernels():
    mesh = plsc.VectorSubcoreMesh(core_axis_name="c", subcore_axis_name="s")
    partition = pl.kernel(
        _partition_body,
        out_type=[
            jax.ShapeDtypeStruct((NW * LISTCAP,), jnp.int32),
            jax.ShapeDtypeStruct((NW * 16,), jnp.int32),
        ],
        mesh=mesh,
        compiler_params=_SC_PARAMS,
        scratch_types=[
            pltpu.VMEM((CH_A,), jnp.int32),
            pltpu.VMEM((CH_A,), jnp.int32),
            pltpu.VMEM((BUFCAP,), jnp.int32),
            pltpu.VMEM((16,), jnp.int32),
        ],
    )
    segmax = pl.kernel(
        _segmax_body,
        out_type=jax.ShapeDtypeStruct((N_PAD * HID,), jnp.float32),
        mesh=mesh,
        compiler_params=_SC_PARAMS,
        scratch_types=[
            pltpu.VMEM(((RANGE + 1) * HID,), jnp.float32),
            pltpu.VMEM((CH_B + 16,), jnp.int32),
            pltpu.VMEM((CH_B,), jnp.int32),
            pltpu.VMEM((CH_B, HID), jnp.float32),
            pltpu.VMEM((16,), jnp.int32),
            pltpu.SemaphoreType.DMA,
        ],
    )
    return partition, segmax


def kernel(x, edge_index, batch, W_emb, b_emb,
           c1_W1, c1_b1, c1_W2, c1_b2,
           c2_W1, c2_b1, c2_W2, c2_b2):
    src = edge_index[0].astype(jnp.int32)
    dst = edge_index[1].astype(jnp.int32)
    batch2 = batch.astype(jnp.int32).reshape(1, N_NODES)
    bemb = b_emb.reshape(1, HID)
    c1b1 = c1_b1.reshape(1, HID)
    c1b2 = c1_b2.reshape(1, HID)
    c2b1 = c2_b1.reshape(1, HID)
    c2b2 = c2_b2.reshape(1, HID)

    _partition, _segmax = _sc_kernels()
    lists, counts = _partition(src, dst)
    h0, m1 = _embed(x, W_emb, bemb, c1_W1, c1b1, c1_W2, c1b2)
    agg1 = _segmax(m1, lists, counts).reshape(N_PAD, HID)[:N_NODES]
    h1, m2 = _combine(h0, agg1, c2_W1, c2b1, c2_W2, c2b2)
    agg2 = _segmax(m2, lists, counts).reshape(N_PAD, HID)[:N_NODES]
    return _final(h1, agg2, batch2)
